# Initial kernel scaffold; baseline (speedup 1.0000x reference)
import jax
import jax.numpy as jnp
from jax.experimental import pallas as pl


def kernel(x, emb, pos_emb):
    raise NotImplementedError("write your pallas kernel here")


# speedup vs baseline: 1.4064x; 1.4064x over previous
import functools

import jax
import jax.numpy as jnp
from jax import lax
from jax.experimental import pallas as pl
from jax.experimental.pallas import tpu as pltpu
from jax.experimental.pallas import tpu_sc as plsc


def _build(total_rows: int, seq: int, hid: int, chunk_seqs: int):
    info = plsc.get_sparse_core_info()
    nc, ns = info.num_cores, info.num_subcores
    nw = nc * ns
    assert total_rows % (nw * seq) == 0
    rows_per_w = total_rows // nw
    ch = chunk_seqs * seq
    assert rows_per_w % ch == 0
    n_chunks = rows_per_w // ch
    assert hid % 16 == 0
    nh = hid // 16

    mesh = plsc.VectorSubcoreMesh(core_axis_name="c", subcore_axis_name="s")

    @functools.partial(
        pl.kernel,
        mesh=mesh,
        compiler_params=pltpu.CompilerParams(use_tc_tiling_on_sc=False),
        out_type=jax.ShapeDtypeStruct((total_rows, hid), jnp.float32),
        scratch_types=[
            pltpu.VMEM((ch,), jnp.int32),
            pltpu.VMEM((ch, hid), jnp.float32),
            pltpu.VMEM((seq, hid), jnp.float32),
            pltpu.SemaphoreType.DMA,
        ],
    )
    def emb_lookup(x_hbm, emb_hbm, pos_hbm, out_hbm, idx_v, rows_v, pos_v, sem):
        wid = lax.axis_index("s") * nc + lax.axis_index("c")
        base = wid * rows_per_w
        pltpu.sync_copy(pos_hbm, pos_v)
        for g in range(n_chunks):
            off = base + g * ch
            pltpu.sync_copy(x_hbm.at[pl.ds(off, ch)], idx_v)
            pltpu.async_copy(emb_hbm.at[idx_v], rows_v, sem).wait()

            def add_pos(s, carry):
                for rep in range(chunk_seqs):
                    r = rep * seq + s
                    for h in range(nh):
                        sl = pl.ds(h * 16, 16)
                        rows_v[r, sl] = rows_v[r, sl] + pos_v[s, sl]
                return carry

            lax.fori_loop(0, seq, add_pos, 0)
            pltpu.sync_copy(rows_v, out_hbm.at[pl.ds(off, ch)])

    return emb_lookup


def kernel(x, emb, pos_emb):
    b, s = x.shape
    hid = emb.shape[1]
    xf = x.reshape(b * s).astype(jnp.int32)
    fn = _build(b * s, s, hid, chunk_seqs=8)
    out = fn(xf, emb, pos_emb)
    return out.reshape(b, s, hid)
